# baseline (speedup 1.0000x reference)
import functools
import math

import jax
import jax.numpy as jnp
from jax import lax
from jax.experimental import pallas as pl
from jax.experimental.pallas import tpu as pltpu
from jax.experimental.pallas import tpu_sc as plsc

_NUM_SAMPLED = 5
_NW = 32
_CHUNK = 128


def _sample_negs(n, num_nodes):
    kk = jnp.arange(num_nodes, dtype=jnp.float32)
    sw = (jnp.log(kk + 2.0) - jnp.log(kk + 1.0)) / math.log(num_nodes + 1)
    sw = sw / jnp.linalg.norm(sw)
    return jax.random.choice(jax.random.key(12345), num_nodes,
                             shape=(n, _NUM_SAMPLED), replace=True, p=sw)


def _sc_gather(weights, idx3):
    nw, c_chunks, chunk = idx3.shape
    d = weights.shape[1]
    rows_total = nw * c_chunks * chunk
    rows_per_w = c_chunks * chunk
    mesh = plsc.VectorSubcoreMesh(core_axis_name="c", subcore_axis_name="s")

    @functools.partial(
        pl.kernel,
        mesh=mesh,
        out_type=jax.ShapeDtypeStruct((rows_total, d), jnp.float32),
        compiler_params=pltpu.CompilerParams(use_tc_tiling_on_sc=False),
        scratch_types=[
            pltpu.VMEM((c_chunks, chunk), jnp.int32),
            pltpu.VMEM((chunk, d), jnp.float32),
            pltpu.VMEM((chunk, d), jnp.float32),
            pltpu.SemaphoreType.DMA,
            pltpu.SemaphoreType.DMA,
        ],
    )
    def k(w_hbm, idx_hbm, out_hbm, idx_v, buf0, buf1, sem0, sem1):
        wid = lax.axis_index("s") * 2 + lax.axis_index("c")
        pltpu.sync_copy(idx_hbm.at[wid], idx_v)
        base = wid * rows_per_w

        def body(p, carry):
            c0 = 2 * p
            c1 = c0 + 1
            h0 = pltpu.async_copy(w_hbm.at[idx_v.at[c0]], buf0, sem0)
            h1 = pltpu.async_copy(w_hbm.at[idx_v.at[c1]], buf1, sem1)
            h0.wait()
            pltpu.sync_copy(buf0, out_hbm.at[pl.ds(base + c0 * chunk, chunk)])
            h1.wait()
            pltpu.sync_copy(buf1, out_hbm.at[pl.ds(base + c1 * chunk, chunk)])
            return carry

        lax.fori_loop(0, c_chunks // 2, body, 0)

    return k(weights, idx3)


def _tc_loss_sum(g6, embs):
    n, d = embs.shape
    bt = 2048
    grid = (n // bt,)

    def body(g_ref, e_ref, o_ref):
        i = pl.program_id(0)
        g = g_ref[...]
        e = e_ref[...]
        s = jnp.sum(g * e[None, :, :], axis=-1)
        slot = lax.broadcasted_iota(jnp.int32, s.shape, 0)
        z = jnp.where(slot == 0, s, -s)
        t = jnp.sum(jnp.log(jax.nn.sigmoid(z))).reshape(1, 1)

        @pl.when(i == 0)
        def _():
            o_ref[...] = jnp.zeros((1, 1), jnp.float32)

        o_ref[...] += t

    out = pl.pallas_call(
        body,
        grid=grid,
        in_specs=[
            pl.BlockSpec((6, bt, d), lambda i: (0, i, 0)),
            pl.BlockSpec((bt, d), lambda i: (i, 0)),
        ],
        out_specs=pl.BlockSpec((1, 1), lambda i: (0, 0)),
        out_shape=jax.ShapeDtypeStruct((1, 1), jnp.float32),
    )(g6, embs)
    return out[0, 0]


def kernel(input, embs, label, weights):
    n, d = embs.shape
    num_nodes = weights.shape[0]
    negs = _sample_negs(n, num_nodes)
    idx = jnp.concatenate([label.astype(jnp.int32),
                           negs.T.reshape(-1).astype(jnp.int32)])
    rows_per_w = idx.shape[0] // _NW
    idx3 = idx.reshape(_NW, rows_per_w // _CHUNK, _CHUNK)
    gathered = _sc_gather(weights, idx3)
    g6 = gathered.reshape(_NUM_SAMPLED + 1, n, d)
    total = _tc_loss_sum(g6, embs)
    return -total / n


# speedup vs baseline: 7.7240x; 7.7240x over previous
import functools
import math

import jax
import jax.numpy as jnp
from jax import lax
from jax.experimental import pallas as pl
from jax.experimental.pallas import tpu as pltpu
from jax.experimental.pallas import tpu_sc as plsc

_NUM_SAMPLED = 5
_NW = 32
_CHUNK = 128


@functools.lru_cache(maxsize=4)
def _sample_negs(n, num_nodes):
    with jax.ensure_compile_time_eval():
        kk = jnp.arange(num_nodes, dtype=jnp.float32)
        sw = (jnp.log(kk + 2.0) - jnp.log(kk + 1.0)) / math.log(num_nodes + 1)
        sw = sw / jnp.linalg.norm(sw)
        negs = jax.random.choice(jax.random.key(12345), num_nodes,
                                 shape=(n, _NUM_SAMPLED), replace=True, p=sw)
        return jax.device_get(negs.T.reshape(-1).astype(jnp.int32))


def _sc_gather(weights, idx3):
    nw, c_chunks, chunk = idx3.shape
    d = weights.shape[1]
    rows_total = nw * c_chunks * chunk
    rows_per_w = c_chunks * chunk
    mesh = plsc.VectorSubcoreMesh(core_axis_name="c", subcore_axis_name="s")

    @functools.partial(
        pl.kernel,
        mesh=mesh,
        out_type=jax.ShapeDtypeStruct((rows_total, d), jnp.float32),
        compiler_params=pltpu.CompilerParams(use_tc_tiling_on_sc=False),
        scratch_types=[
            pltpu.VMEM((c_chunks, chunk), jnp.int32),
            pltpu.VMEM((chunk, d), jnp.float32),
            pltpu.VMEM((chunk, d), jnp.float32),
            pltpu.SemaphoreType.DMA,
            pltpu.SemaphoreType.DMA,
        ],
    )
    def k(w_hbm, idx_hbm, out_hbm, idx_v, buf0, buf1, sem0, sem1):
        wid = lax.axis_index("s") * 2 + lax.axis_index("c")
        pltpu.sync_copy(idx_hbm.at[wid], idx_v)
        base = wid * rows_per_w

        def body(p, carry):
            c0 = 2 * p
            c1 = c0 + 1
            h0 = pltpu.async_copy(w_hbm.at[idx_v.at[c0]], buf0, sem0)
            h1 = pltpu.async_copy(w_hbm.at[idx_v.at[c1]], buf1, sem1)
            h0.wait()
            pltpu.sync_copy(buf0, out_hbm.at[pl.ds(base + c0 * chunk, chunk)])
            h1.wait()
            pltpu.sync_copy(buf1, out_hbm.at[pl.ds(base + c1 * chunk, chunk)])
            return carry

        lax.fori_loop(0, c_chunks // 2, body, 0)

    return k(weights, idx3)


def _tc_loss_sum(g6, embs):
    n, d = embs.shape
    bt = 2048
    grid = (n // bt,)

    def body(g_ref, e_ref, o_ref):
        i = pl.program_id(0)
        g = g_ref[...]
        e = e_ref[...]
        s = jnp.sum(g * e[None, :, :], axis=-1)
        slot = lax.broadcasted_iota(jnp.int32, s.shape, 0)
        z = jnp.where(slot == 0, s, -s)
        t = jnp.sum(jnp.log(jax.nn.sigmoid(z))).reshape(1, 1)

        @pl.when(i == 0)
        def _():
            o_ref[...] = jnp.zeros((1, 1), jnp.float32)

        o_ref[...] += t

    out = pl.pallas_call(
        body,
        grid=grid,
        in_specs=[
            pl.BlockSpec((6, bt, d), lambda i: (0, i, 0)),
            pl.BlockSpec((bt, d), lambda i: (i, 0)),
        ],
        out_specs=pl.BlockSpec((1, 1), lambda i: (0, 0)),
        out_shape=jax.ShapeDtypeStruct((1, 1), jnp.float32),
    )(g6, embs)
    return out[0, 0]


def kernel(input, embs, label, weights):
    n, d = embs.shape
    num_nodes = weights.shape[0]
    negs_flat = _sample_negs(n, num_nodes)
    idx = jnp.concatenate([label.astype(jnp.int32), jnp.asarray(negs_flat)])
    rows_per_w = idx.shape[0] // _NW
    idx3 = idx.reshape(_NW, rows_per_w // _CHUNK, _CHUNK)
    gathered = _sc_gather(weights, idx3)
    g6 = gathered.reshape(_NUM_SAMPLED + 1, n, d)
    total = _tc_loss_sum(g6, embs)
    return -total / n


# speedup vs baseline: 7.9694x; 1.0318x over previous
import functools
import math

import jax
import jax.numpy as jnp
from jax import lax
from jax.experimental import pallas as pl
from jax.experimental.pallas import tpu as pltpu
from jax.experimental.pallas import tpu_sc as plsc

_NUM_SAMPLED = 5
_NW = 32
_CHUNK = 128


@functools.lru_cache(maxsize=4)
def _sample_negs(n, num_nodes):
    with jax.ensure_compile_time_eval():
        kk = jnp.arange(num_nodes, dtype=jnp.float32)
        sw = (jnp.log(kk + 2.0) - jnp.log(kk + 1.0)) / math.log(num_nodes + 1)
        sw = sw / jnp.linalg.norm(sw)
        negs = jax.random.choice(jax.random.key(12345), num_nodes,
                                 shape=(n, _NUM_SAMPLED), replace=True, p=sw)
        return jax.device_get(negs.T.reshape(-1).astype(jnp.int32))


def _sc_scores(weights, embs, idx3):
    nw, c_chunks, chunk = idx3.shape
    n, d = embs.shape
    bb = n // nw
    sub = bb // chunk
    groups = chunk // 16
    mesh = plsc.VectorSubcoreMesh(core_axis_name="c", subcore_axis_name="s")

    @functools.partial(
        pl.kernel,
        mesh=mesh,
        out_type=jax.ShapeDtypeStruct((nw, c_chunks * chunk), jnp.float32),
        compiler_params=pltpu.CompilerParams(use_tc_tiling_on_sc=False,
                                             needs_layout_passes=False),
        scratch_types=[
            pltpu.VMEM((c_chunks, chunk), jnp.int32),
            pltpu.VMEM((bb, d), jnp.float32),
            pltpu.VMEM((chunk, d), jnp.float32),
            pltpu.VMEM((chunk, d), jnp.float32),
            pltpu.VMEM((c_chunks * chunk,), jnp.float32),
            pltpu.SemaphoreType.DMA,
            pltpu.SemaphoreType.DMA,
        ],
    )
    def k(w_hbm, e_hbm, idx_hbm, out_hbm, idx_v, e_v, bufa, bufb, s_v,
          sema, semb):
        wid = lax.axis_index("s") * 2 + lax.axis_index("c")
        pltpu.sync_copy(idx_hbm.at[wid], idx_v)
        pltpu.sync_copy(e_hbm.at[pl.ds(wid * bb, bb)], e_v)
        lanes = lax.iota(jnp.int32, 16)

        def compute(c, buf):
            j = lax.rem(c, sub)
            for g in range(groups):
                wrow = g * 16 + lanes
                erow = j * chunk + g * 16 + lanes
                acc = jnp.zeros((16,), jnp.float32)
                for dd in range(d):
                    col = jnp.full((16,), dd, jnp.int32)
                    wv = plsc.load_gather(buf, [wrow, col])
                    ev = plsc.load_gather(e_v, [erow, col])
                    acc = acc + wv * ev
                s_v[pl.ds(c * chunk + g * 16, 16)] = acc

        def fire(c, buf, sem):
            return pltpu.async_copy(w_hbm.at[idx_v.at[c]], buf, sem)

        def wait(buf, sem):
            pltpu.make_async_copy(w_hbm.at[idx_v.at[0]], buf, sem).wait()

        fire(0, bufa, sema)

        def body(p, carry):
            c0 = 2 * p
            fire(c0 + 1, bufb, semb)
            wait(bufa, sema)
            compute(c0, bufa)

            @pl.when(p < c_chunks // 2 - 1)
            def _():
                fire(c0 + 2, bufa, sema)

            wait(bufb, semb)
            compute(c0 + 1, bufb)
            return carry

        lax.fori_loop(0, c_chunks // 2, body, 0)
        pltpu.sync_copy(s_v, out_hbm.at[wid])

    return k(weights, embs, idx3)


def _tc_loss_sum(scores, pos_cols):
    def body(s_ref, o_ref):
        s = s_ref[...]
        col = lax.broadcasted_iota(jnp.int32, s.shape, 1)
        z = jnp.where(col < pos_cols, s, -s)
        o_ref[...] = jnp.sum(jnp.log(jax.nn.sigmoid(z))).reshape(1, 1)

    out = pl.pallas_call(
        body,
        out_shape=jax.ShapeDtypeStruct((1, 1), jnp.float32),
    )(scores)
    return out[0, 0]


def kernel(input, embs, label, weights):
    n, d = embs.shape
    num_nodes = weights.shape[0]
    negs_flat = _sample_negs(n, num_nodes)
    idx = jnp.concatenate([label.astype(jnp.int32), jnp.asarray(negs_flat)])
    bb = n // _NW
    idx3 = (idx.reshape(_NUM_SAMPLED + 1, _NW, bb // _CHUNK, _CHUNK)
            .transpose(1, 0, 2, 3)
            .reshape(_NW, -1, _CHUNK))
    scores = _sc_scores(weights, embs, idx3)
    total = _tc_loss_sum(scores, bb)
    return -total / n
